# baseline (speedup 1.0000x reference)
import functools

import jax
import jax.numpy as jnp
from jax import lax
from jax.experimental import pallas as pl
from jax.experimental.pallas import tpu as pltpu
from jax.experimental.pallas import tpu_sc as plsc

C = 128
CP = C // 2
E = 160000
NUM_WORKERS = 32
GATHER_CHUNK = 400
TC_BLOCK = 3200


def _sc_gather(table, idx_flat):
    n_rows = idx_flat.shape[0]
    per_w = n_rows // NUM_WORKERS
    n_pairs = per_w // (2 * GATHER_CHUNK)
    mesh = plsc.VectorSubcoreMesh(core_axis_name="c", subcore_axis_name="s")

    @functools.partial(
        pl.kernel,
        mesh=mesh,
        out_type=jax.ShapeDtypeStruct((n_rows, CP), jnp.int32),
        compiler_params=pltpu.CompilerParams(use_tc_tiling_on_sc=False),
        scratch_types=[
            pltpu.VMEM((per_w,), jnp.int32),
            pltpu.VMEM((GATHER_CHUNK, CP), jnp.int32),
            pltpu.VMEM((GATHER_CHUNK, CP), jnp.int32),
            pltpu.SemaphoreType.DMA,
            pltpu.SemaphoreType.DMA,
        ],
    )
    def gather_kernel(table_hbm, idx_hbm, out_hbm, idx_v, buf_a, buf_b, sem_a, sem_b):
        wid = lax.axis_index("s") * 2 + lax.axis_index("c")
        base = wid * per_w
        pltpu.sync_copy(idx_hbm.at[pl.ds(base, per_w)], idx_v)

        def body(g, carry):
            off_a = g * (2 * GATHER_CHUNK)
            off_b = off_a + GATHER_CHUNK
            cp_a = pltpu.async_copy(
                table_hbm.at[idx_v.at[pl.ds(off_a, GATHER_CHUNK)]], buf_a, sem_a
            )
            cp_b = pltpu.async_copy(
                table_hbm.at[idx_v.at[pl.ds(off_b, GATHER_CHUNK)]], buf_b, sem_b
            )
            cp_a.wait()
            pltpu.sync_copy(buf_a, out_hbm.at[pl.ds(base + off_a, GATHER_CHUNK)])
            cp_b.wait()
            pltpu.sync_copy(buf_b, out_hbm.at[pl.ds(base + off_b, GATHER_CHUNK)])
            return carry

        lax.fori_loop(0, n_pairs, body, 0)

    return gather_kernel(table, idx_flat)


def _conv_pass(x0_src, nb, wstack):
    n_blocks = E // TC_BLOCK
    dn = (((1,), (1,)), ((), ()))

    def body(x0_ref, nb_ref, w_ref, y_ref, s1_ref, s2_ref):
        i = pl.program_id(0)
        x0 = x0_ref[...]
        a0 = nb_ref[0]
        a1 = nb_ref[1]
        a2 = nb_ref[2]
        a3 = nb_ref[3]
        feats = (x0, a0 + a2, a1 + a3, jnp.abs(a0 - a2), jnp.abs(a1 - a3))
        y = jnp.zeros((C, TC_BLOCK), jnp.float32)
        for w_idx, f in enumerate(feats):
            y = y + jax.lax.dot_general(
                w_ref[w_idx], f, dn, preferred_element_type=jnp.float32
            )
        y_ref[...] = y.astype(jnp.bfloat16)

        @pl.when(i == 0)
        def _():
            s1_ref[...] = jnp.zeros_like(s1_ref)
            s2_ref[...] = jnp.zeros_like(s2_ref)

        s1_ref[...] += jnp.sum(y, axis=1, keepdims=True)
        s2_ref[...] += jnp.sum(y * y, axis=1, keepdims=True)

    return pl.pallas_call(
        body,
        grid=(n_blocks,),
        in_specs=[
            pl.BlockSpec((TC_BLOCK, C), lambda i: (i, 0)),
            pl.BlockSpec((4, TC_BLOCK, C), lambda i: (0, i, 0)),
            pl.BlockSpec((5, C, C), lambda i: (0, 0, 0)),
        ],
        out_specs=[
            pl.BlockSpec((C, TC_BLOCK), lambda i: (0, i)),
            pl.BlockSpec((C, 1), lambda i: (0, 0)),
            pl.BlockSpec((C, 1), lambda i: (0, 0)),
        ],
        out_shape=[
            jax.ShapeDtypeStruct((C, E), jnp.bfloat16),
            jax.ShapeDtypeStruct((C, 1), jnp.float32),
            jax.ShapeDtypeStruct((C, 1), jnp.float32),
        ],
    )(x0_src, nb, wstack)


def _norm_pass(y, s1, s2):
    n_blocks = E // TC_BLOCK
    inv_e = 1.0 / E

    def body(y_ref, s1_ref, s2_ref, o_ref):
        mean = s1_ref[...] * inv_e
        var = s2_ref[...] * inv_e - mean * mean
        inv = lax.rsqrt(var + 1e-5)
        o_ref[...] = jnp.maximum((y_ref[...].astype(jnp.float32) - mean) * inv, 0.0)

    return pl.pallas_call(
        body,
        grid=(n_blocks,),
        in_specs=[
            pl.BlockSpec((C, TC_BLOCK), lambda i: (0, i)),
            pl.BlockSpec((C, 1), lambda i: (0, 0)),
            pl.BlockSpec((C, 1), lambda i: (0, 0)),
        ],
        out_specs=pl.BlockSpec((C, TC_BLOCK), lambda i: (0, i)),
        out_shape=jax.ShapeDtypeStruct((C, E), jnp.float32),
    )(y, s1, s2)


def kernel(fe, edge_index, W, b):
    del b
    table_bf = fe[0].astype(jnp.bfloat16).T
    table_i32 = lax.bitcast_convert_type(
        table_bf.reshape(E, CP, 2), jnp.int32
    )
    idx_flat = edge_index[0].T.reshape(-1)
    nb_i32 = _sc_gather(table_i32, idx_flat)
    nb_bf = lax.bitcast_convert_type(nb_i32, jnp.bfloat16).reshape(4, E, C)
    wstack = jnp.moveaxis(W[:, :, 0, :], -1, 0).astype(jnp.bfloat16)
    y, s1, s2 = _conv_pass(table_bf, nb_bf, wstack)
    out = _norm_pass(y, s1, s2)
    return out[None]
